# Initial kernel scaffold; baseline (speedup 1.0000x reference)
import jax
import jax.numpy as jnp
from jax.experimental import pallas as pl


def kernel(h, edge_index, gcn_W, gcn_b, ln1_g, ln1_b, ln2_g, ln2_b, ln3_g, ln3_b, Wq, Wk, Wv, bq, bk, bv, Wo, bo, W1, b1, W2, b2):
    raise NotImplementedError("write your pallas kernel here")


# speedup vs baseline: 2.8633x; 2.8633x over previous
import functools

import jax
import jax.numpy as jnp
from jax.experimental import pallas as pl
from jax.experimental.pallas import tpu as pltpu

N = 10000
D = 128
H = 2
DH = D // H
E = 320000
NPAD = 10240
BLK = 256
NBLK = NPAD // BLK
EPS = 1e-5


def _ln(x, g, b):
    m = jnp.mean(x, axis=-1, keepdims=True)
    v = jnp.mean((x - m) ** 2, axis=-1, keepdims=True)
    return (x - m) * jax.lax.rsqrt(v + EPS) * g + b


def _dotT(x, w):
    return jax.lax.dot_general(x, w, (((1,), (1,)), ((), ())),
                               preferred_element_type=jnp.float32)


def _pre_body(h_ref, g_ref, b_ref, w_ref, o_ref):
    x = _ln(h_ref[...], g_ref[...], b_ref[...])
    o_ref[...] = _dotT(x, w_ref[...])


_row_spec = pl.BlockSpec((BLK, D), lambda i: (i, 0))
_full_vec = pl.BlockSpec((D,), lambda i: (0,))
_full_mat = pl.BlockSpec((D, D), lambda i: (0, 0))

_k_pre = pl.pallas_call(
    _pre_body,
    grid=(NBLK,),
    in_specs=[_row_spec, _full_vec, _full_vec, _full_mat],
    out_specs=_row_spec,
    out_shape=jax.ShapeDtypeStruct((NPAD, D), jnp.float32),
)


def _scale_body(xw_ref, degp_ref, y_ref, dinvb_ref):
    parts = degp_ref[...]
    deg = parts[0, :] + parts[1, :] + 1.0
    dinv = jax.lax.rsqrt(deg)[:, None]
    dinvb = jnp.broadcast_to(dinv, (BLK, D))
    dinvb_ref[...] = dinvb
    y_ref[...] = xw_ref[...] * dinvb


_k_scale = pl.pallas_call(
    _scale_body,
    grid=(NBLK,),
    in_specs=[_row_spec, pl.BlockSpec((2, BLK), lambda i: (0, i))],
    out_specs=[_row_spec, _row_spec],
    out_shape=[jax.ShapeDtypeStruct((NPAD, D), jnp.float32),
               jax.ShapeDtypeStruct((NPAD, D), jnp.float32)],
)


def _qkv_body(h_ref, y_ref, s0_ref, s1_ref, dinvb_ref, gb_ref,
              g2_ref, b2_ref, wq_ref, wk_ref, wv_ref,
              bq_ref, bk_ref, bv_ref,
              h1_ref, q_ref, k_ref, v_ref):
    h1 = (h_ref[...] + gb_ref[...]
          + dinvb_ref[...] * (s0_ref[...] + s1_ref[...] + y_ref[...]))
    h1_ref[...] = h1
    x = _ln(h1, g2_ref[...], b2_ref[...])
    q = _dotT(x, wq_ref[...]) + bq_ref[...]
    k = _dotT(x, wk_ref[...]) + bk_ref[...]
    v = _dotT(x, wv_ref[...]) + bv_ref[...]
    q_ref[0, ...] = q[:, :DH]
    q_ref[1, ...] = q[:, DH:]
    k_ref[0, ...] = k[:, :DH]
    k_ref[1, ...] = k[:, DH:]
    v_ref[0, ...] = v[:, :DH]
    v_ref[1, ...] = v[:, DH:]


_k_qkv = pl.pallas_call(
    _qkv_body,
    grid=(NBLK,),
    in_specs=[_row_spec, _row_spec, _row_spec, _row_spec, _row_spec,
              _full_vec, _full_vec, _full_vec,
              _full_mat, _full_mat, _full_mat,
              _full_vec, _full_vec, _full_vec],
    out_specs=[_row_spec] + [pl.BlockSpec((H, BLK, DH), lambda i: (0, i, 0))] * 3,
    out_shape=([jax.ShapeDtypeStruct((NPAD, D), jnp.float32)]
               + [jax.ShapeDtypeStruct((H, NPAD, DH), jnp.float32)] * 3),
)


def _attn_body(q_ref, k_ref, v_ref, o_ref):
    q = q_ref[0]
    k = k_ref[0]
    s = jax.lax.dot_general(q, k, (((1,), (1,)), ((), ())),
                            preferred_element_type=jnp.float32) * 0.125
    col = jax.lax.broadcasted_iota(jnp.int32, s.shape, 1)
    s = jnp.where(col < N, s, -1e30)
    m = jnp.max(s, axis=-1, keepdims=True)
    p = jnp.exp(s - m)
    l = jnp.sum(p, axis=-1, keepdims=True)
    o = jnp.dot(p, v_ref[0], preferred_element_type=jnp.float32)
    o_ref[0, ...] = o / l


_k_attn = pl.pallas_call(
    _attn_body,
    grid=(H, NBLK),
    in_specs=[pl.BlockSpec((1, BLK, DH), lambda h, i: (h, i, 0)),
              pl.BlockSpec((1, NPAD, DH), lambda h, i: (h, 0, 0)),
              pl.BlockSpec((1, NPAD, DH), lambda h, i: (h, 0, 0))],
    out_specs=pl.BlockSpec((1, BLK, DH), lambda h, i: (h, i, 0)),
    out_shape=jax.ShapeDtypeStruct((H, NPAD, DH), jnp.float32),
)


def _post_body(h1_ref, a_ref, wo_ref, bo_ref, g3_ref, b3_ref,
               w1_ref, b1_ref, w2_ref, b2_ref, o_ref):
    a = jnp.concatenate([a_ref[0], a_ref[1]], axis=-1)
    h2 = h1_ref[...] + _dotT(a, wo_ref[...]) + bo_ref[...]
    x = _ln(h2, g3_ref[...], b3_ref[...])
    t = jnp.maximum(_dotT(x, w1_ref[...]) + b1_ref[...], 0.0)
    o_ref[...] = h2 + _dotT(t, w2_ref[...]) + b2_ref[...]


_k_post = pl.pallas_call(
    _post_body,
    grid=(NBLK,),
    in_specs=[_row_spec, pl.BlockSpec((H, BLK, DH), lambda i: (0, i, 0)),
              _full_mat, _full_vec, _full_vec, _full_vec,
              pl.BlockSpec((2 * D, D), lambda i: (0, 0)),
              pl.BlockSpec((2 * D,), lambda i: (0,)),
              pl.BlockSpec((D, 2 * D), lambda i: (0, 0)),
              _full_vec],
    out_specs=_row_spec,
    out_shape=jax.ShapeDtypeStruct((NPAD, D), jnp.float32),
)


def kernel(h, edge_index, gcn_W, gcn_b, ln1_g, ln1_b, ln2_g, ln2_b, ln3_g,
           ln3_b, Wq, Wk, Wv, bq, bk, bv, Wo, bo, W1, b1, W2, b2):
    hp = jnp.pad(h, ((0, NPAD - N), (0, 0)))
    src = edge_index[0]
    dst = edge_index[1]

    deg = jnp.zeros((NPAD,), jnp.float32).at[dst].add(1.0)
    degp = jnp.stack([deg, jnp.zeros((NPAD,), jnp.float32)])

    xw = _k_pre(hp, ln1_g, ln1_b, gcn_W)
    y, dinvb = _k_scale(xw, degp)

    s = jnp.zeros((NPAD, D), jnp.float32).at[dst].add(y[src])
    s0, s1 = s, jnp.zeros((NPAD, D), jnp.float32)

    h1, q, k, v = _k_qkv(hp, y, s0, s1, dinvb, gcn_b, ln2_g, ln2_b,
                         Wq, Wk, Wv, bq, bk, bv)
    attn = _k_attn(q, k, v)
    out = _k_post(h1, attn, Wo, bo, ln3_g, ln3_b, W1, b1, W2, b2)
    return out[:N]


# speedup vs baseline: 8.1796x; 2.8567x over previous
import dataclasses
import functools

import jax
import jax.numpy as jnp
from jax import lax
from jax.experimental import pallas as pl
from jax.experimental.pallas import tpu as pltpu
from jax.experimental.pallas import tpu_sc as plsc

N = 10000
D = 128
H = 2
DH = D // H
E = 320000
NPAD = 10240
BLK = 256
NBLK = NPAD // BLK
EPS = 1e-5

NC = 2
NS = 16
NW = NC * NS
EP = E // NW
C = 125
NCHUNK = EP // C
RPT = NPAD // NS

_sc_mesh = plsc.VectorSubcoreMesh(core_axis_name="c", subcore_axis_name="s")
_sc_cp = pltpu.CompilerParams()
if "needs_layout_passes" in pltpu.CompilerParams.__dataclass_fields__:
    _sc_cp = dataclasses.replace(_sc_cp, needs_layout_passes=False)


@functools.partial(
    pl.kernel,
    out_type=jax.ShapeDtypeStruct((NC, NPAD), jnp.float32),
    mesh=_sc_mesh,
    compiler_params=_sc_cp,
    scratch_types=[pltpu.VMEM((EP,), jnp.int32),
                   pltpu.VMEM((NPAD,), jnp.float32),
                   pltpu.VMEM((NS, RPT), jnp.float32),
                   pltpu.VMEM((RPT,), jnp.float32),
                   pltpu.VMEM_SHARED((NS, NPAD), jnp.float32)],
)
def _k_deg(dst_hbm, degp_hbm, dst_v, deg_v, blk_v, acc_v, shared):
    c = lax.axis_index("c")
    s = lax.axis_index("s")
    wid = c * NS + s
    pltpu.sync_copy(dst_hbm.at[wid], dst_v)

    @pl.loop(0, NPAD, step=16)
    def _(i):
        deg_v[pl.ds(i, 16)] = jnp.zeros((16,), jnp.float32)

    ones = jnp.ones((16,), jnp.float32)

    @pl.loop(0, EP, step=16)
    def _(e):
        plsc.addupdate_scatter(deg_v, [dst_v[pl.ds(e, 16)]], ones)

    pltpu.sync_copy(deg_v, shared.at[s])
    plsc.subcore_barrier()
    pltpu.sync_copy(shared.at[:, pl.ds(s * RPT, RPT)], blk_v)

    @pl.loop(0, RPT, step=16)
    def _(i):
        tot = blk_v[0, pl.ds(i, 16)]
        for j in range(1, NS):
            tot = tot + blk_v[j, pl.ds(i, 16)]
        acc_v[pl.ds(i, 16)] = tot

    pltpu.sync_copy(acc_v, degp_hbm.at[c, pl.ds(s * RPT, RPT)])


@functools.partial(
    pl.kernel,
    out_type=jax.ShapeDtypeStruct((NC, NPAD, D), jnp.float32),
    mesh=_sc_mesh,
    compiler_params=_sc_cp,
    scratch_types=[pltpu.VMEM((NCHUNK, C), jnp.int32),
                   pltpu.VMEM((NCHUNK, C), jnp.int32),
                   pltpu.VMEM((C, D), jnp.float32),
                   pltpu.VMEM((64, D), jnp.float32),
                   pltpu.VMEM_SHARED((NPAD, D), jnp.float32),
                   pltpu.SemaphoreType.DMA],
)
def _k_msg(src_hbm, dst_hbm, y_hbm, sp_hbm,
           src_v, dst_v, rows_v, z_v, shared, sem):
    c = lax.axis_index("c")
    s = lax.axis_index("s")
    wid = c * NS + s
    pltpu.sync_copy(src_hbm.at[wid], src_v)
    pltpu.sync_copy(dst_hbm.at[wid], dst_v)

    @pl.loop(0, 64)
    def _(i):
        @pl.loop(0, D, step=16)
        def _(j):
            z_v[i, pl.ds(j, 16)] = jnp.zeros((16,), jnp.float32)

    @pl.loop(0, RPT, step=64)
    def _(t):
        pltpu.sync_copy(z_v, shared.at[pl.ds(s * RPT + t, 64)])

    plsc.subcore_barrier()

    @pl.loop(0, NCHUNK)
    def _(j):
        pltpu.async_copy(y_hbm.at[src_v.at[j]], rows_v, sem).wait()
        pltpu.sync_copy(rows_v, shared.at[dst_v.at[j]], add=True)

    plsc.subcore_barrier()
    pltpu.sync_copy(shared.at[pl.ds(s * RPT, RPT)],
                    sp_hbm.at[c, pl.ds(s * RPT, RPT)])


def _ln(x, g, b):
    m = jnp.mean(x, axis=-1, keepdims=True)
    v = jnp.mean((x - m) ** 2, axis=-1, keepdims=True)
    return (x - m) * jax.lax.rsqrt(v + EPS) * g + b


def _dotT(x, w):
    return jax.lax.dot_general(x, w, (((1,), (1,)), ((), ())),
                               preferred_element_type=jnp.float32)


def _pre_body(h_ref, g_ref, b_ref, w_ref, o_ref):
    x = _ln(h_ref[...], g_ref[...], b_ref[...])
    o_ref[...] = _dotT(x, w_ref[...])


_row_spec = pl.BlockSpec((BLK, D), lambda i: (i, 0))
_full_vec = pl.BlockSpec((D,), lambda i: (0,))
_full_mat = pl.BlockSpec((D, D), lambda i: (0, 0))

_k_pre = pl.pallas_call(
    _pre_body,
    grid=(NBLK,),
    in_specs=[_row_spec, _full_vec, _full_vec, _full_mat],
    out_specs=_row_spec,
    out_shape=jax.ShapeDtypeStruct((NPAD, D), jnp.float32),
)


def _scale_body(xw_ref, degp_ref, y_ref, dinvb_ref):
    parts = degp_ref[...]
    deg = parts[0, :] + parts[1, :] + 1.0
    dinv = jax.lax.rsqrt(deg)[:, None]
    dinvb = jnp.broadcast_to(dinv, (BLK, D))
    dinvb_ref[...] = dinvb
    y_ref[...] = xw_ref[...] * dinvb


_k_scale = pl.pallas_call(
    _scale_body,
    grid=(NBLK,),
    in_specs=[_row_spec, pl.BlockSpec((2, BLK), lambda i: (0, i))],
    out_specs=[_row_spec, _row_spec],
    out_shape=[jax.ShapeDtypeStruct((NPAD, D), jnp.float32),
               jax.ShapeDtypeStruct((NPAD, D), jnp.float32)],
)


def _qkv_body(h_ref, y_ref, s0_ref, s1_ref, dinvb_ref, gb_ref,
              g2_ref, b2_ref, wq_ref, wk_ref, wv_ref,
              bq_ref, bk_ref, bv_ref,
              h1_ref, q_ref, k_ref, v_ref):
    h1 = (h_ref[...] + gb_ref[...]
          + dinvb_ref[...] * (s0_ref[...] + s1_ref[...] + y_ref[...]))
    h1_ref[...] = h1
    x = _ln(h1, g2_ref[...], b2_ref[...])
    q = _dotT(x, wq_ref[...]) + bq_ref[...]
    k = _dotT(x, wk_ref[...]) + bk_ref[...]
    v = _dotT(x, wv_ref[...]) + bv_ref[...]
    q_ref[0, ...] = q[:, :DH]
    q_ref[1, ...] = q[:, DH:]
    k_ref[0, ...] = k[:, :DH]
    k_ref[1, ...] = k[:, DH:]
    v_ref[0, ...] = v[:, :DH]
    v_ref[1, ...] = v[:, DH:]


_k_qkv = pl.pallas_call(
    _qkv_body,
    grid=(NBLK,),
    in_specs=[_row_spec, _row_spec, _row_spec, _row_spec, _row_spec,
              _full_vec, _full_vec, _full_vec,
              _full_mat, _full_mat, _full_mat,
              _full_vec, _full_vec, _full_vec],
    out_specs=[_row_spec] + [pl.BlockSpec((H, BLK, DH), lambda i: (0, i, 0))] * 3,
    out_shape=([jax.ShapeDtypeStruct((NPAD, D), jnp.float32)]
               + [jax.ShapeDtypeStruct((H, NPAD, DH), jnp.float32)] * 3),
)


def _attn_body(q_ref, k_ref, v_ref, o_ref):
    q = q_ref[0]
    k = k_ref[0]
    s = jax.lax.dot_general(q, k, (((1,), (1,)), ((), ())),
                            preferred_element_type=jnp.float32) * 0.125
    col = jax.lax.broadcasted_iota(jnp.int32, s.shape, 1)
    s = jnp.where(col < N, s, -1e30)
    m = jnp.max(s, axis=-1, keepdims=True)
    p = jnp.exp(s - m)
    l = jnp.sum(p, axis=-1, keepdims=True)
    o = jnp.dot(p, v_ref[0], preferred_element_type=jnp.float32)
    o_ref[0, ...] = o / l


_k_attn = pl.pallas_call(
    _attn_body,
    grid=(H, NBLK),
    in_specs=[pl.BlockSpec((1, BLK, DH), lambda h, i: (h, i, 0)),
              pl.BlockSpec((1, NPAD, DH), lambda h, i: (h, 0, 0)),
              pl.BlockSpec((1, NPAD, DH), lambda h, i: (h, 0, 0))],
    out_specs=pl.BlockSpec((1, BLK, DH), lambda h, i: (h, i, 0)),
    out_shape=jax.ShapeDtypeStruct((H, NPAD, DH), jnp.float32),
)


def _post_body(h1_ref, a_ref, wo_ref, bo_ref, g3_ref, b3_ref,
               w1_ref, b1_ref, w2_ref, b2_ref, o_ref):
    a = jnp.concatenate([a_ref[0], a_ref[1]], axis=-1)
    h2 = h1_ref[...] + _dotT(a, wo_ref[...]) + bo_ref[...]
    x = _ln(h2, g3_ref[...], b3_ref[...])
    t = jnp.maximum(_dotT(x, w1_ref[...]) + b1_ref[...], 0.0)
    o_ref[...] = h2 + _dotT(t, w2_ref[...]) + b2_ref[...]


_k_post = pl.pallas_call(
    _post_body,
    grid=(NBLK,),
    in_specs=[_row_spec, pl.BlockSpec((H, BLK, DH), lambda i: (0, i, 0)),
              _full_mat, _full_vec, _full_vec, _full_vec,
              pl.BlockSpec((2 * D, D), lambda i: (0, 0)),
              pl.BlockSpec((2 * D,), lambda i: (0,)),
              pl.BlockSpec((D, 2 * D), lambda i: (0, 0)),
              _full_vec],
    out_specs=_row_spec,
    out_shape=jax.ShapeDtypeStruct((NPAD, D), jnp.float32),
)


def kernel(h, edge_index, gcn_W, gcn_b, ln1_g, ln1_b, ln2_g, ln2_b, ln3_g,
           ln3_b, Wq, Wk, Wv, bq, bk, bv, Wo, bo, W1, b1, W2, b2):
    hp = jnp.pad(h, ((0, NPAD - N), (0, 0)))
    src2 = edge_index[0].reshape(NW, NCHUNK, C)
    dst1 = edge_index[1].reshape(NW, EP)
    dst2 = edge_index[1].reshape(NW, NCHUNK, C)

    degp = _k_deg(dst1)
    xw = _k_pre(hp, ln1_g, ln1_b, gcn_W)
    y, dinvb = _k_scale(xw, degp)
    sp = _k_msg(src2, dst2, y)

    h1, q, k, v = _k_qkv(hp, y, sp[0], sp[1], dinvb, gcn_b, ln2_g, ln2_b,
                         Wq, Wk, Wv, bq, bk, bv)
    attn = _k_attn(q, k, v)
    out = _k_post(h1, attn, Wo, bo, ln3_g, ln3_b, W1, b1, W2, b2)
    return out[:N]


# speedup vs baseline: 10.2399x; 1.2519x over previous
import dataclasses
import functools

import jax
import jax.numpy as jnp
from jax import lax
from jax.experimental import pallas as pl
from jax.experimental.pallas import tpu as pltpu
from jax.experimental.pallas import tpu_sc as plsc

N = 10000
D = 128
H = 2
DH = D // H
E = 320000
NPAD = 10240
BLK = 256
NBLK = NPAD // BLK
EPS = 1e-5

NC = 2
NS = 16
NW = NC * NS
EP = E // NW
C = 125
NCHUNK = EP // C
RPT = NPAD // NS

_sc_mesh = plsc.VectorSubcoreMesh(core_axis_name="c", subcore_axis_name="s")
_sc_cp = pltpu.CompilerParams()
if "needs_layout_passes" in pltpu.CompilerParams.__dataclass_fields__:
    _sc_cp = dataclasses.replace(_sc_cp, needs_layout_passes=False)


@functools.partial(
    pl.kernel,
    out_type=jax.ShapeDtypeStruct((NC, NPAD), jnp.float32),
    mesh=_sc_mesh,
    compiler_params=_sc_cp,
    scratch_types=[pltpu.VMEM((EP,), jnp.int32),
                   pltpu.VMEM((NPAD,), jnp.float32),
                   pltpu.VMEM((NS, RPT), jnp.float32),
                   pltpu.VMEM((RPT,), jnp.float32),
                   pltpu.VMEM_SHARED((NS, NPAD), jnp.float32)],
)
def _k_deg(dst_hbm, degp_hbm, dst_v, deg_v, blk_v, acc_v, shared):
    c = lax.axis_index("c")
    s = lax.axis_index("s")
    wid = c * NS + s
    pltpu.sync_copy(dst_hbm.at[wid], dst_v)

    @pl.loop(0, NPAD, step=16)
    def _(i):
        deg_v[pl.ds(i, 16)] = jnp.zeros((16,), jnp.float32)

    ones = jnp.ones((16,), jnp.float32)

    @pl.loop(0, EP, step=16)
    def _(e):
        plsc.addupdate_scatter(deg_v, [dst_v[pl.ds(e, 16)]], ones)

    pltpu.sync_copy(deg_v, shared.at[s])
    plsc.subcore_barrier()
    pltpu.sync_copy(shared.at[:, pl.ds(s * RPT, RPT)], blk_v)

    @pl.loop(0, RPT, step=16)
    def _(i):
        tot = blk_v[0, pl.ds(i, 16)]
        for j in range(1, NS):
            tot = tot + blk_v[j, pl.ds(i, 16)]
        acc_v[pl.ds(i, 16)] = tot

    pltpu.sync_copy(acc_v, degp_hbm.at[c, pl.ds(s * RPT, RPT)])


@functools.partial(
    pl.kernel,
    out_type=jax.ShapeDtypeStruct((NC, NPAD, D), jnp.float32),
    mesh=_sc_mesh,
    compiler_params=_sc_cp,
    scratch_types=[pltpu.VMEM((NCHUNK, C), jnp.int32),
                   pltpu.VMEM((NCHUNK, C), jnp.int32),
                   pltpu.VMEM((C, D), jnp.float32),
                   pltpu.VMEM((64, D), jnp.float32),
                   pltpu.VMEM_SHARED((NPAD, D), jnp.float32),
                   pltpu.SemaphoreType.DMA],
)
def _k_msg(src_hbm, dst_hbm, y_hbm, sp_hbm,
           src_v, dst_v, rows_v, z_v, shared, sem):
    c = lax.axis_index("c")
    s = lax.axis_index("s")
    wid = c * NS + s
    pltpu.sync_copy(src_hbm.at[wid], src_v)
    pltpu.sync_copy(dst_hbm.at[wid], dst_v)

    @pl.loop(0, 64)
    def _(i):
        @pl.loop(0, D, step=16)
        def _(j):
            z_v[i, pl.ds(j, 16)] = jnp.zeros((16,), jnp.float32)

    @pl.loop(0, RPT, step=64)
    def _(t):
        pltpu.sync_copy(z_v, shared.at[pl.ds(s * RPT + t, 64)])

    plsc.subcore_barrier()

    @pl.loop(0, NCHUNK)
    def _(j):
        pltpu.async_copy(y_hbm.at[src_v.at[j]], rows_v, sem).wait()
        pltpu.sync_copy(rows_v, shared.at[dst_v.at[j]], add=True)

    plsc.subcore_barrier()
    pltpu.sync_copy(shared.at[pl.ds(s * RPT, RPT)],
                    sp_hbm.at[c, pl.ds(s * RPT, RPT)])


def _ln(x, g, b):
    m = jnp.mean(x, axis=-1, keepdims=True)
    v = jnp.mean((x - m) ** 2, axis=-1, keepdims=True)
    return (x - m) * jax.lax.rsqrt(v + EPS) * g + b


def _dotT(x, w):
    return jax.lax.dot_general(x, w, (((1,), (1,)), ((), ())),
                               preferred_element_type=jnp.float32)


def _pre_body(h_ref, g_ref, b_ref, w_ref, o_ref):
    x = _ln(h_ref[...], g_ref[...], b_ref[...])
    o_ref[...] = _dotT(x, w_ref[...])


_row_spec = pl.BlockSpec((BLK, D), lambda i: (i, 0))
_full_vec = pl.BlockSpec((D,), lambda i: (0,))
_full_mat = pl.BlockSpec((D, D), lambda i: (0, 0))

_k_pre = pl.pallas_call(
    _pre_body,
    grid=(NBLK,),
    in_specs=[_row_spec, _full_vec, _full_vec, _full_mat],
    out_specs=_row_spec,
    out_shape=jax.ShapeDtypeStruct((NPAD, D), jnp.float32),
)


def _scale_body(xw_ref, degp_ref, y_ref, dinvb_ref):
    parts = degp_ref[...]
    deg = parts[0, :] + parts[1, :] + 1.0
    dinv = jax.lax.rsqrt(deg)[:, None]
    dinvb = jnp.broadcast_to(dinv, (BLK, D))
    dinvb_ref[...] = dinvb
    y_ref[...] = xw_ref[...] * dinvb


_k_scale = pl.pallas_call(
    _scale_body,
    grid=(NBLK,),
    in_specs=[_row_spec, pl.BlockSpec((2, BLK), lambda i: (0, i))],
    out_specs=[_row_spec, _row_spec],
    out_shape=[jax.ShapeDtypeStruct((NPAD, D), jnp.float32),
               jax.ShapeDtypeStruct((NPAD, D), jnp.float32)],
)


def _qkv_body(h_ref, y_ref, s0_ref, s1_ref, dinvb_ref, gb_ref,
              g2_ref, b2_ref, wq_ref, wk_ref, wv_ref,
              bq_ref, bk_ref, bv_ref,
              h1_ref, q_ref, k_ref, v_ref):
    h1 = (h_ref[...] + gb_ref[...]
          + dinvb_ref[...] * (s0_ref[...] + s1_ref[...] + y_ref[...]))
    h1_ref[...] = h1
    x = _ln(h1, g2_ref[...], b2_ref[...])
    q = _dotT(x, wq_ref[...]) + bq_ref[...]
    k = _dotT(x, wk_ref[...]) + bk_ref[...]
    v = _dotT(x, wv_ref[...]) + bv_ref[...]
    q = q.astype(jnp.bfloat16)
    k = k.astype(jnp.bfloat16)
    v = v.astype(jnp.bfloat16)
    q_ref[0, ...] = q[:, :DH]
    q_ref[1, ...] = q[:, DH:]
    k_ref[0, ...] = k[:, :DH]
    k_ref[1, ...] = k[:, DH:]
    v_ref[0, ...] = v[:, :DH]
    v_ref[1, ...] = v[:, DH:]


_k_qkv = pl.pallas_call(
    _qkv_body,
    grid=(NBLK,),
    in_specs=[_row_spec, _row_spec, _row_spec, _row_spec, _row_spec,
              _full_vec, _full_vec, _full_vec,
              _full_mat, _full_mat, _full_mat,
              _full_vec, _full_vec, _full_vec],
    out_specs=[_row_spec] + [pl.BlockSpec((H, BLK, DH), lambda i: (0, i, 0))] * 3,
    out_shape=([jax.ShapeDtypeStruct((NPAD, D), jnp.float32)]
               + [jax.ShapeDtypeStruct((H, NPAD, DH), jnp.bfloat16)] * 3),
)


def _attn_body(q_ref, k_ref, v_ref, o_ref):
    q = q_ref[0]
    k = k_ref[0]
    s = jax.lax.dot_general(q, k, (((1,), (1,)), ((), ())),
                            preferred_element_type=jnp.float32) * 0.125
    col = jax.lax.broadcasted_iota(jnp.int32, s.shape, 1)
    s = jnp.where(col < N, s, -1e30)
    m = jnp.max(s, axis=-1, keepdims=True)
    p = jnp.exp(s - m)
    l = jnp.sum(p, axis=-1, keepdims=True)
    o = jnp.dot(p.astype(jnp.bfloat16), v_ref[0],
                preferred_element_type=jnp.float32)
    o_ref[0, ...] = o / l


_k_attn = pl.pallas_call(
    _attn_body,
    grid=(H, NBLK),
    in_specs=[pl.BlockSpec((1, BLK, DH), lambda h, i: (h, i, 0)),
              pl.BlockSpec((1, NPAD, DH), lambda h, i: (h, 0, 0)),
              pl.BlockSpec((1, NPAD, DH), lambda h, i: (h, 0, 0))],
    out_specs=pl.BlockSpec((1, BLK, DH), lambda h, i: (h, i, 0)),
    out_shape=jax.ShapeDtypeStruct((H, NPAD, DH), jnp.float32),
)


def _post_body(h1_ref, a_ref, wo_ref, bo_ref, g3_ref, b3_ref,
               w1_ref, b1_ref, w2_ref, b2_ref, o_ref):
    a = jnp.concatenate([a_ref[0], a_ref[1]], axis=-1)
    h2 = h1_ref[...] + _dotT(a, wo_ref[...]) + bo_ref[...]
    x = _ln(h2, g3_ref[...], b3_ref[...])
    t = jnp.maximum(_dotT(x, w1_ref[...]) + b1_ref[...], 0.0)
    o_ref[...] = h2 + _dotT(t, w2_ref[...]) + b2_ref[...]


_k_post = pl.pallas_call(
    _post_body,
    grid=(NBLK,),
    in_specs=[_row_spec, pl.BlockSpec((H, BLK, DH), lambda i: (0, i, 0)),
              _full_mat, _full_vec, _full_vec, _full_vec,
              pl.BlockSpec((2 * D, D), lambda i: (0, 0)),
              pl.BlockSpec((2 * D,), lambda i: (0,)),
              pl.BlockSpec((D, 2 * D), lambda i: (0, 0)),
              _full_vec],
    out_specs=_row_spec,
    out_shape=jax.ShapeDtypeStruct((NPAD, D), jnp.float32),
)


def kernel(h, edge_index, gcn_W, gcn_b, ln1_g, ln1_b, ln2_g, ln2_b, ln3_g,
           ln3_b, Wq, Wk, Wv, bq, bk, bv, Wo, bo, W1, b1, W2, b2):
    hp = jnp.pad(h, ((0, NPAD - N), (0, 0)))
    src2 = edge_index[0].reshape(NW, NCHUNK, C)
    dst1 = edge_index[1].reshape(NW, EP)
    dst2 = edge_index[1].reshape(NW, NCHUNK, C)

    degp = _k_deg(dst1)
    xw = _k_pre(hp, ln1_g, ln1_b, gcn_W)
    y, dinvb = _k_scale(xw, degp)
    sp = _k_msg(src2, dst2, y)

    h1, q, k, v = _k_qkv(hp, y, sp[0], sp[1], dinvb, gcn_b, ln2_g, ln2_b,
                         Wq, Wk, Wv, bq, bk, bv)
    attn = _k_attn(q, k, v)
    out = _k_post(h1, attn, Wo, bo, ln3_g, ln3_b, W1, b1, W2, b2)
    return out[:N]
